# baseline (speedup 1.0000x reference)
import functools

import jax
import jax.numpy as jnp
from jax import lax
from jax.experimental import pallas as pl
from jax.experimental.pallas import tpu as pltpu
from jax.experimental.pallas import tpu_sc as plsc

_EPS = 0.1
_ALPHA = 0.2
_ROWS_PER_BLOCK = 8

_NUM_CORES = 2
_NUM_SUBCORES = 16
_NUM_WORKERS = _NUM_CORES * _NUM_SUBCORES


def _sc_gather_body(C, BPW, x_hbm, pid_hbm, g0_hbm, g1_hbm, g2_hbm,
                    pid_v, idx_v, val_v, sem):
    wid = lax.axis_index("s") * _NUM_CORES + lax.axis_index("c")
    base = wid * BPW
    pltpu.sync_copy(pid_hbm.at[pl.ds(base, BPW)], pid_v)
    outs = (g0_hbm, g1_hbm, g2_hbm)
    for k in range(3):
        for h in range(BPW // 16):
            rows = base + h * 16 + lax.iota(jnp.int32, 16)
            p = pid_v[pl.ds(h * 16, 16)]
            idx_v[pl.ds(h * 16, 16)] = rows * C + p * 3 + k
        pltpu.async_copy(x_hbm.at[idx_v], val_v, sem).wait()
        pltpu.sync_copy(val_v, outs[k].at[pl.ds(base, BPW)])


def _sc_gather(x_flat, pids, B, C):
    BPW = B // _NUM_WORKERS
    mesh = plsc.VectorSubcoreMesh(core_axis_name="c", subcore_axis_name="s")
    f = functools.partial(
        pl.kernel,
        mesh=mesh,
        out_type=[jax.ShapeDtypeStruct((B,), jnp.float32)] * 3,
        scratch_types=[
            pltpu.VMEM((BPW,), jnp.int32),
            pltpu.VMEM((BPW,), jnp.int32),
            pltpu.VMEM((BPW,), jnp.float32),
            pltpu.SemaphoreType.DMA,
        ],
    )(functools.partial(_sc_gather_body, C, BPW))
    return f(x_flat, pids)


def _loss_body(x_ref, g0_ref, g1_ref, g2_ref, vid_ref, out_ref):
    i = pl.program_id(0)
    n = pl.num_programs(0)
    x = x_ref[...]
    C = x.shape[1]
    B_total = n * x.shape[0]

    m = jnp.max(x, axis=1, keepdims=True)
    z = jnp.sum(jnp.exp(x - m), axis=1, keepdims=True)
    s = jnp.sum(x, axis=1, keepdims=True)

    g0 = g0_ref[...]
    g1 = g1_ref[...]
    g2 = g2_ref[...]
    vid = vid_ref[...]
    gt = jnp.where(vid == 0, g0, jnp.where(vid == 1, g1, g2))

    c = m + jnp.log(z)
    ep1 = jnp.exp(g0 - c) + jnp.exp(g1 - c) + jnp.exp(g2 - c)
    ep2 = jnp.exp(gt - c)
    L = s - C * c
    L3 = (g0 + g1 + g2) - 3.0 * c
    lpt = gt - c
    e1 = _ALPHA * (1.0 - ep1)
    e2 = _ALPHA * (1.0 - ep2)
    S = (e1 / (C - 3)) * (L - L3) + 0.5 * e2 * (L3 - lpt) + (1.0 - e1 - e2) * lpt
    contrib = (1.0 - _EPS) * S + (_EPS / C) * L
    bs = jnp.sum(contrib, axis=0, keepdims=True)

    @pl.when(i == 0)
    def _():
        out_ref[...] = jnp.zeros_like(out_ref)

    out_ref[...] += bs

    @pl.when(i == n - 1)
    def _():
        out_ref[...] = out_ref[...] * (-1.0 / B_total)


@jax.jit
def kernel(inputs, pids, vids):
    B, C = inputs.shape
    R = _ROWS_PER_BLOCK
    grid = B // R
    pids32 = pids.astype(jnp.int32)
    g0, g1, g2 = _sc_gather(inputs.reshape(B * C), pids32, B, C)
    vids2 = vids.reshape(B, 1).astype(jnp.int32)
    out = pl.pallas_call(
        _loss_body,
        grid=(grid,),
        in_specs=[
            pl.BlockSpec((R, C), lambda i: (i, 0)),
            pl.BlockSpec((R, 1), lambda i: (i, 0)),
            pl.BlockSpec((R, 1), lambda i: (i, 0)),
            pl.BlockSpec((R, 1), lambda i: (i, 0)),
            pl.BlockSpec((R, 1), lambda i: (i, 0)),
        ],
        out_specs=pl.BlockSpec((1, 1), lambda i: (0, 0)),
        out_shape=jax.ShapeDtypeStruct((1, 1), jnp.float32),
    )(inputs, g0.reshape(B, 1), g1.reshape(B, 1), g2.reshape(B, 1), vids2)
    return out[0, 0]


# speedup vs baseline: 7.9239x; 7.9239x over previous
import functools

import jax
import jax.numpy as jnp
from jax import lax
from jax.experimental import pallas as pl
from jax.experimental.pallas import tpu as pltpu
from jax.experimental.pallas import tpu_sc as plsc

_EPS = 0.1
_ALPHA = 0.2
_ROWS_PER_BLOCK = 8
_W = 256

_NUM_CORES = 2
_NUM_SUBCORES = 16
_NUM_WORKERS = _NUM_CORES * _NUM_SUBCORES


def _tail_start(C):
    return ((C - 1) // 128) * 128


def _window_start(p3, C):
    return jnp.minimum(p3 & -128, _tail_start(C) - _W)


def _sc_windows_body(C, BPW, x_hbm, pid_hbm, win_hbm, pid_s, win_v, sem):
    wid = lax.axis_index("s") * _NUM_CORES + lax.axis_index("c")
    base = wid * BPW
    pltpu.sync_copy(pid_hbm.at[pl.ds(base, BPW)], pid_s)
    tmax = (_tail_start(C) - _W) >> 7
    copies = []
    for r in range(BPW):
        chunk = pid_s[pl.ds((r // 16) * 16, 16)]
        t = jnp.minimum((chunk[r % 16] * 3) >> 7, tmax)
        start = t * 128
        row8 = base + (r & -8)
        copies.append(
            pltpu.async_copy(x_hbm.at[pl.ds(row8, 8), pl.ds(start, _W)],
                             win_v.at[pl.ds(r * 8, 8)], sem))
    for cp in copies:
        cp.wait()
    pltpu.sync_copy(win_v, win_hbm.at[pl.ds(base * 8, BPW * 8)])


def _sc_windows(x, pids, B, C):
    BPW = B // _NUM_WORKERS
    mesh = plsc.VectorSubcoreMesh(core_axis_name="c", subcore_axis_name="s")
    f = functools.partial(
        pl.kernel,
        mesh=mesh,
        out_type=jax.ShapeDtypeStruct((B * 8, _W), jnp.float32),
        scratch_types=[
            pltpu.VMEM((BPW,), jnp.int32),
            pltpu.VMEM((BPW * 8, _W), jnp.float32),
            pltpu.SemaphoreType.DMA,
        ],
    )(functools.partial(_sc_windows_body, C, BPW))
    return f(x, pids)


def _loss_body(x_ref, win_ref, pid_ref, vid_ref, out_ref):
    i = pl.program_id(0)
    n = pl.num_programs(0)
    x = x_ref[...]
    R, C = x.shape
    B_total = n * R

    m = jnp.max(x, axis=1, keepdims=True)
    z = jnp.sum(jnp.exp(x - m), axis=1, keepdims=True)
    s = jnp.sum(x, axis=1, keepdims=True)

    win8 = win_ref[...]
    r0 = lax.broadcasted_iota(jnp.int32, win8.shape, 0)
    r1 = lax.broadcasted_iota(jnp.int32, win8.shape, 1)
    win = jnp.sum(jnp.where(r0 == r1, win8, jnp.zeros_like(win8)), axis=1)
    p3 = pid_ref[...] * 3
    vid = vid_ref[...]
    A = _tail_start(C)
    tail = lax.slice(x, (0, A), (R, C))
    o = p3 - _window_start(p3, C)
    colw = lax.broadcasted_iota(jnp.int32, win.shape, 1)
    colt = lax.broadcasted_iota(jnp.int32, tail.shape, 1) + A
    zw = jnp.zeros_like(win)
    zt = jnp.zeros_like(tail)

    def pick(q, qo):
        gw = jnp.sum(jnp.where(colw == qo, win, zw), axis=1, keepdims=True)
        gt_ = jnp.sum(jnp.where(colt == q, tail, zt), axis=1, keepdims=True)
        return gw + gt_

    g0 = pick(p3, o)
    g1 = pick(p3 + 1, o + 1)
    g2 = pick(p3 + 2, o + 2)
    gv = pick(p3 + vid, o + vid)

    c = m + jnp.log(z)
    ep1 = jnp.exp(g0 - c) + jnp.exp(g1 - c) + jnp.exp(g2 - c)
    ep2 = jnp.exp(gv - c)
    L = s - C * c
    L3 = (g0 + g1 + g2) - 3.0 * c
    lpt = gv - c
    e1 = _ALPHA * (1.0 - ep1)
    e2 = _ALPHA * (1.0 - ep2)
    S = (e1 / (C - 3)) * (L - L3) + 0.5 * e2 * (L3 - lpt) + (1.0 - e1 - e2) * lpt
    contrib = (1.0 - _EPS) * S + (_EPS / C) * L
    bs = jnp.sum(contrib, axis=0, keepdims=True)

    @pl.when(i == 0)
    def _():
        out_ref[...] = jnp.zeros_like(out_ref)

    out_ref[...] += bs

    @pl.when(i == n - 1)
    def _():
        out_ref[...] = out_ref[...] * (-1.0 / B_total)


@jax.jit
def kernel(inputs, pids, vids):
    B, C = inputs.shape
    R = _ROWS_PER_BLOCK
    grid = B // R
    pids32 = pids.astype(jnp.int32)
    win = _sc_windows(inputs, pids32, B, C).reshape(B, 8, _W)
    pids2 = pids32.reshape(B, 1)
    vids2 = vids.reshape(B, 1).astype(jnp.int32)
    out = pl.pallas_call(
        _loss_body,
        grid=(grid,),
        in_specs=[
            pl.BlockSpec((R, C), lambda i: (i, 0)),
            pl.BlockSpec((R, 8, _W), lambda i: (i, 0, 0)),
            pl.BlockSpec((R, 1), lambda i: (i, 0)),
            pl.BlockSpec((R, 1), lambda i: (i, 0)),
        ],
        out_specs=pl.BlockSpec((1, 1), lambda i: (0, 0)),
        out_shape=jax.ShapeDtypeStruct((1, 1), jnp.float32),
    )(inputs, win, pids2, vids2)
    return out[0, 0]


# speedup vs baseline: 8.5630x; 1.0807x over previous
import functools

import jax
import jax.numpy as jnp
from jax import lax
from jax.experimental import pallas as pl
from jax.experimental.pallas import tpu as pltpu
from jax.experimental.pallas import tpu_sc as plsc

_EPS = 0.1
_ALPHA = 0.2
_ROWS_PER_BLOCK = 8
_W = 256

_NUM_CORES = 2
_NUM_SUBCORES = 16
_NUM_WORKERS = _NUM_CORES * _NUM_SUBCORES


def _tail_start(C):
    return ((C - 1) // 128) * 128


def _window_start(p3, C):
    return jnp.minimum(p3 & -128, _tail_start(C) - _W)


def _sc_windows_body(C, BPW, x_hbm, pid_hbm, win_hbm, pid_s, win_v, sem):
    wid = lax.axis_index("s") * _NUM_CORES + lax.axis_index("c")
    base = wid * BPW
    pltpu.sync_copy(pid_hbm.at[pl.ds(base, BPW)], pid_s)
    tmax = (_tail_start(C) - _W) >> 7
    copies = []
    for r in range(BPW):
        chunk = pid_s[pl.ds((r // 16) * 16, 16)]
        t = jnp.minimum((chunk[r % 16] * 3) >> 7, tmax)
        start = t * 128
        row8 = base + (r & -8)
        copies.append(
            pltpu.async_copy(x_hbm.at[pl.ds(row8, 8), pl.ds(start, _W)],
                             win_v.at[pl.ds(r * 8, 8)], sem))
    for cp in copies:
        cp.wait()
    pltpu.sync_copy(win_v, win_hbm.at[pl.ds(base * 8, BPW * 8)])


def _sc_windows(x, pids, B, C):
    BPW = B // _NUM_WORKERS
    mesh = plsc.VectorSubcoreMesh(core_axis_name="c", subcore_axis_name="s")
    f = functools.partial(
        pl.kernel,
        mesh=mesh,
        out_type=jax.ShapeDtypeStruct((B * 8, _W), jnp.float32),
        scratch_types=[
            pltpu.VMEM((BPW,), jnp.int32),
            pltpu.VMEM((BPW * 8, _W), jnp.float32),
            pltpu.SemaphoreType.DMA,
        ],
    )(functools.partial(_sc_windows_body, C, BPW))
    return f(x, pids)


def _loss_body(x_ref, win_ref, pid_ref, vid_ref, out_ref):
    i = pl.program_id(0)
    n = pl.num_programs(0)
    x = x_ref[...]
    R, C = x.shape
    B_total = n * R

    m = jnp.max(x, axis=1, keepdims=True)
    z = jnp.sum(jnp.exp(x - m), axis=1, keepdims=True)
    s = jnp.sum(x, axis=1, keepdims=True)

    win8 = win_ref[...]
    R8 = win8.shape[0]
    e0 = lax.broadcasted_iota(jnp.int32, (R, R8), 0)
    e1 = lax.broadcasted_iota(jnp.int32, (R, R8), 1)
    diag = (e1 == 9 * e0).astype(jnp.float32)
    win = jax.lax.dot_general(diag, win8, (((1,), (0,)), ((), ())),
                              preferred_element_type=jnp.float32)
    p3 = pid_ref[...] * 3
    vid = vid_ref[...]
    A = _tail_start(C)
    tail = lax.slice(x, (0, A), (R, C))
    o = p3 - _window_start(p3, C)
    colw = lax.broadcasted_iota(jnp.int32, win.shape, 1)
    colt = lax.broadcasted_iota(jnp.int32, tail.shape, 1) + A
    zw = jnp.zeros_like(win)
    zt = jnp.zeros_like(tail)

    def pick(q, qo):
        gw = jnp.sum(jnp.where(colw == qo, win, zw), axis=1, keepdims=True)
        gt_ = jnp.sum(jnp.where(colt == q, tail, zt), axis=1, keepdims=True)
        return gw + gt_

    g0 = pick(p3, o)
    g1 = pick(p3 + 1, o + 1)
    g2 = pick(p3 + 2, o + 2)
    gv = pick(p3 + vid, o + vid)

    c = m + jnp.log(z)
    ep1 = jnp.exp(g0 - c) + jnp.exp(g1 - c) + jnp.exp(g2 - c)
    ep2 = jnp.exp(gv - c)
    L = s - C * c
    L3 = (g0 + g1 + g2) - 3.0 * c
    lpt = gv - c
    e1 = _ALPHA * (1.0 - ep1)
    e2 = _ALPHA * (1.0 - ep2)
    S = (e1 / (C - 3)) * (L - L3) + 0.5 * e2 * (L3 - lpt) + (1.0 - e1 - e2) * lpt
    contrib = (1.0 - _EPS) * S + (_EPS / C) * L
    bs = jnp.sum(contrib, axis=0, keepdims=True)

    @pl.when(i == 0)
    def _():
        out_ref[...] = jnp.zeros_like(out_ref)

    out_ref[...] += bs

    @pl.when(i == n - 1)
    def _():
        out_ref[...] = out_ref[...] * (-1.0 / B_total)


@jax.jit
def kernel(inputs, pids, vids):
    B, C = inputs.shape
    R = _ROWS_PER_BLOCK
    grid = B // R
    pids32 = pids.astype(jnp.int32)
    win = _sc_windows(inputs, pids32, B, C)
    pids2 = pids32.reshape(B, 1)
    vids2 = vids.reshape(B, 1).astype(jnp.int32)
    out = pl.pallas_call(
        _loss_body,
        grid=(grid,),
        in_specs=[
            pl.BlockSpec((R, C), lambda i: (i, 0)),
            pl.BlockSpec((R * 8, _W), lambda i: (i, 0)),
            pl.BlockSpec((R, 1), lambda i: (i, 0)),
            pl.BlockSpec((R, 1), lambda i: (i, 0)),
        ],
        out_specs=pl.BlockSpec((1, 1), lambda i: (0, 0)),
        out_shape=jax.ShapeDtypeStruct((1, 1), jnp.float32),
    )(inputs, win, pids2, vids2)
    return out[0, 0]


# speedup vs baseline: 24.3437x; 2.8429x over previous
import functools

import jax
import jax.numpy as jnp
from jax import lax
from jax.experimental import pallas as pl
from jax.experimental.pallas import tpu as pltpu
from jax.experimental.pallas import tpu_sc as plsc

_EPS = 0.1
_ALPHA = 0.2
_CB = 1024
_RB = 128
_SLAB = 16

_NUM_CORES = 2
_NUM_SUBCORES = 16
_NUM_WORKERS = _NUM_CORES * _NUM_SUBCORES


def _slab_start(p3, C):
    return jnp.minimum(p3 >> 3, (C - _SLAB) >> 3) * 8


def _sc_slabs_body(C, BPW, x_hbm, pid_hbm, win_hbm, pid_v, win_v, sem):
    wid = lax.axis_index("s") * _NUM_CORES + lax.axis_index("c")
    base = wid * BPW
    pltpu.sync_copy(pid_hbm.at[pl.ds(base, BPW)], pid_v)
    copies = []
    for r in range(BPW):
        chunk = pid_v[pl.ds((r // 16) * 16, 16)]
        p3 = chunk[r % 16] * 3
        start = _slab_start(p3, C)
        cg = ((base + r) >> 7) * 128
        copies.append(
            pltpu.async_copy(x_hbm.at[pl.ds(start, _SLAB), pl.ds(cg, 128)],
                             win_v.at[r], sem))
    for cp in copies:
        cp.wait()
    pltpu.sync_copy(win_v, win_hbm.at[pl.ds(base, BPW)])


def _sc_slabs(xt, pids, B, C):
    BPW = B // _NUM_WORKERS
    mesh = plsc.VectorSubcoreMesh(core_axis_name="c", subcore_axis_name="s")
    f = functools.partial(
        pl.kernel,
        mesh=mesh,
        out_type=jax.ShapeDtypeStruct((B, _SLAB, 128), jnp.float32),
        scratch_types=[
            pltpu.VMEM((BPW,), jnp.int32),
            pltpu.VMEM((BPW, _SLAB, 128), jnp.float32),
            pltpu.SemaphoreType.DMA,
        ],
    )(functools.partial(_sc_slabs_body, C, BPW))
    return f(xt, pids)


def _stream_body(C, x_ref, m_out, z_out, s_out, tail_out, macc, zacc, sacc):
    i = pl.program_id(0)
    n = pl.num_programs(0)
    x = x_ref[...]
    Cb, B = x.shape

    @pl.when(i == 0)
    def _():
        macc[...] = jnp.full_like(macc, -jnp.inf)
        zacc[...] = jnp.zeros_like(zacc)
        sacc[...] = jnp.zeros_like(sacc)

    def update(xv, xm):
        bm = jnp.max(xm, axis=0, keepdims=True)
        m_new = jnp.maximum(macc[...], bm)
        scale = jnp.exp(macc[...] - m_new)
        zacc[...] = zacc[...] * scale + jnp.sum(
            jnp.exp(xm - m_new), axis=0, keepdims=True)
        sacc[...] += jnp.sum(xv, axis=0, keepdims=True)
        macc[...] = m_new

    @pl.when(i < n - 1)
    def _():
        update(x, x)

    @pl.when(i == n - 1)
    def _():
        row = lax.broadcasted_iota(jnp.int32, x.shape, 0) + i * Cb
        valid = row < C
        xv = jnp.where(valid, x, jnp.zeros_like(x))
        xm = jnp.where(valid, x, jnp.full_like(x, -jnp.inf))
        update(xv, xm)
        m_out[...] = macc[...]
        z_out[...] = zacc[...]
        s_out[...] = sacc[...]
        lo = C - 8 - (n - 1) * Cb
        tail_out[...] = lax.slice(x, (lo, 0), (lo + 8, B))


def _stream(xt, B, C):
    n = (C + _CB - 1) // _CB
    return pl.pallas_call(
        functools.partial(_stream_body, C),
        grid=(n,),
        in_specs=[pl.BlockSpec((_CB, B), lambda i: (i, 0))],
        out_specs=[
            pl.BlockSpec((1, B), lambda i: (0, 0)),
            pl.BlockSpec((1, B), lambda i: (0, 0)),
            pl.BlockSpec((1, B), lambda i: (0, 0)),
            pl.BlockSpec((8, B), lambda i: (0, 0)),
        ],
        out_shape=[
            jax.ShapeDtypeStruct((1, B), jnp.float32),
            jax.ShapeDtypeStruct((1, B), jnp.float32),
            jax.ShapeDtypeStruct((1, B), jnp.float32),
            jax.ShapeDtypeStruct((8, B), jnp.float32),
        ],
        scratch_shapes=[
            pltpu.VMEM((1, B), jnp.float32),
            pltpu.VMEM((1, B), jnp.float32),
            pltpu.VMEM((1, B), jnp.float32),
        ],
    )(xt)


def _combine_body(C, m_ref, z_ref, s_ref, tail_ref, win_ref, pid_ref, vid_ref,
                  out_ref):
    i = pl.program_id(0)
    n = pl.num_programs(0)
    m = m_ref[...]
    z = z_ref[...]
    s = s_ref[...]
    win = win_ref[...]
    tail = tail_ref[...]
    p3 = pid_ref[...] * 3
    vid = vid_ref[...]

    d0 = lax.broadcasted_iota(jnp.int32, win.shape, 0)
    d2 = lax.broadcasted_iota(jnp.int32, win.shape, 2)
    wcol = jnp.sum(jnp.where(d2 == d0, win, jnp.zeros_like(win)), axis=2)

    start = _slab_start(p3, C)
    rowg = lax.broadcasted_iota(jnp.int32, wcol.shape, 1) + start
    rowt = lax.broadcasted_iota(jnp.int32, tail.shape, 1) + (C - 8)
    A = ((C - _SLAB) >> 3 << 3) + _SLAB
    zs = jnp.zeros_like(wcol)
    zt = jnp.zeros_like(tail)

    def pick(q):
        gw = jnp.sum(jnp.where(rowg == q, wcol, zs), axis=1, keepdims=True)
        gt_ = jnp.sum(jnp.where((rowt == q) & (rowt >= A), tail, zt),
                      axis=1, keepdims=True)
        return gw + gt_

    g0 = pick(p3)
    g1 = pick(p3 + 1)
    g2 = pick(p3 + 2)
    gv = pick(p3 + vid)

    c = m + jnp.log(z)
    ep1 = jnp.exp(g0 - c) + jnp.exp(g1 - c) + jnp.exp(g2 - c)
    ep2 = jnp.exp(gv - c)
    L = s - C * c
    L3 = (g0 + g1 + g2) - 3.0 * c
    lpt = gv - c
    e1 = _ALPHA * (1.0 - ep1)
    e2 = _ALPHA * (1.0 - ep2)
    S = (e1 / (C - 3)) * (L - L3) + 0.5 * e2 * (L3 - lpt) + (1.0 - e1 - e2) * lpt
    contrib = (1.0 - _EPS) * S + (_EPS / C) * L
    bs = jnp.sum(contrib, axis=0, keepdims=True)

    @pl.when(i == 0)
    def _():
        out_ref[...] = jnp.zeros_like(out_ref)

    out_ref[...] += bs

    @pl.when(i == n - 1)
    def _():
        B_total = n * m.shape[0]
        out_ref[...] = out_ref[...] * (-1.0 / B_total)


def _combine(m, z, s, tail_t, win, pids2, vids2, B, C):
    n = B // _RB
    return pl.pallas_call(
        functools.partial(_combine_body, C),
        grid=(n,),
        in_specs=[
            pl.BlockSpec((_RB, 1), lambda i: (i, 0)),
            pl.BlockSpec((_RB, 1), lambda i: (i, 0)),
            pl.BlockSpec((_RB, 1), lambda i: (i, 0)),
            pl.BlockSpec((_RB, 8), lambda i: (i, 0)),
            pl.BlockSpec((_RB, _SLAB, 128), lambda i: (i, 0, 0)),
            pl.BlockSpec((_RB, 1), lambda i: (i, 0)),
            pl.BlockSpec((_RB, 1), lambda i: (i, 0)),
        ],
        out_specs=pl.BlockSpec((1, 1), lambda i: (0, 0)),
        out_shape=jax.ShapeDtypeStruct((1, 1), jnp.float32),
    )(m, z, s, tail_t, win, pids2, vids2)


@jax.jit
def kernel(inputs, pids, vids):
    B, C = inputs.shape
    xt = inputs.T
    pids32 = pids.astype(jnp.int32)
    win = _sc_slabs(xt, pids32, B, C)
    m, z, s, tail = _stream(xt, B, C)
    out = _combine(m.reshape(B, 1), z.reshape(B, 1), s.reshape(B, 1),
                   tail.T, win, pids32.reshape(B, 1),
                   vids.reshape(B, 1).astype(jnp.int32), B, C)
    return out[0, 0]


# speedup vs baseline: 27.2377x; 1.1189x over previous
import functools

import jax
import jax.numpy as jnp
from jax import lax
from jax.experimental import pallas as pl
from jax.experimental.pallas import tpu as pltpu
from jax.experimental.pallas import tpu_sc as plsc

_EPS = 0.1
_ALPHA = 0.2
_CB = 2048
_RB = 128
_SLAB = 16

_NUM_CORES = 2
_NUM_SUBCORES = 16
_NUM_WORKERS = _NUM_CORES * _NUM_SUBCORES


def _slab_start(p3, C):
    return jnp.minimum(p3 >> 3, (C - _SLAB) >> 3) * 8


def _sc_slabs_body(C, BPW, x_hbm, pid_hbm, win_hbm, pid_v, win_v, sem):
    wid = lax.axis_index("s") * _NUM_CORES + lax.axis_index("c")
    base = wid * BPW
    pltpu.sync_copy(pid_hbm.at[pl.ds(base, BPW)], pid_v)
    copies = []
    for r in range(BPW):
        chunk = pid_v[pl.ds((r // 16) * 16, 16)]
        p3 = chunk[r % 16] * 3
        start = _slab_start(p3, C)
        cg = ((base + r) >> 7) * 128
        copies.append(
            pltpu.async_copy(x_hbm.at[pl.ds(start, _SLAB), pl.ds(cg, 128)],
                             win_v.at[r], sem))
    for cp in copies:
        cp.wait()
    pltpu.sync_copy(win_v, win_hbm.at[pl.ds(base, BPW)])


def _sc_slabs(xt, pids, B, C):
    BPW = B // _NUM_WORKERS
    mesh = plsc.VectorSubcoreMesh(core_axis_name="c", subcore_axis_name="s")
    f = functools.partial(
        pl.kernel,
        mesh=mesh,
        out_type=jax.ShapeDtypeStruct((B, _SLAB, 128), jnp.float32),
        scratch_types=[
            pltpu.VMEM((BPW,), jnp.int32),
            pltpu.VMEM((BPW, _SLAB, 128), jnp.float32),
            pltpu.SemaphoreType.DMA,
        ],
    )(functools.partial(_sc_slabs_body, C, BPW))
    return f(xt, pids)


def _stream_body(C, x_ref, m_out, z_out, s_out, tail_out, macc, zacc, sacc):
    i = pl.program_id(0)
    n = pl.num_programs(0)
    x = x_ref[...]
    Cb, B = x.shape

    @pl.when(i == 0)
    def _():
        macc[...] = jnp.full_like(macc, -jnp.inf)
        zacc[...] = jnp.zeros_like(zacc)
        sacc[...] = jnp.zeros_like(sacc)

    def update(xv, xm):
        bm = jnp.max(xm, axis=0, keepdims=True)
        m_new = jnp.maximum(macc[...], bm)
        scale = jnp.exp(macc[...] - m_new)
        zacc[...] = zacc[...] * scale + jnp.sum(
            jnp.exp(xm - m_new), axis=0, keepdims=True)
        sacc[...] += jnp.sum(xv, axis=0, keepdims=True)
        macc[...] = m_new

    @pl.when(i < n - 1)
    def _():
        update(x, x)

    @pl.when(i == n - 1)
    def _():
        row = lax.broadcasted_iota(jnp.int32, x.shape, 0) + i * Cb
        valid = row < C
        xv = jnp.where(valid, x, jnp.zeros_like(x))
        xm = jnp.where(valid, x, jnp.full_like(x, -jnp.inf))
        update(xv, xm)
        m_out[...] = macc[...]
        z_out[...] = zacc[...]
        s_out[...] = sacc[...]
        lo = C - 8 - (n - 1) * Cb
        tail_out[...] = lax.slice(x, (lo, 0), (lo + 8, B))


def _stream(xt, B, C):
    n = (C + _CB - 1) // _CB
    return pl.pallas_call(
        functools.partial(_stream_body, C),
        grid=(n,),
        in_specs=[pl.BlockSpec((_CB, B), lambda i: (i, 0))],
        out_specs=[
            pl.BlockSpec((1, B), lambda i: (0, 0)),
            pl.BlockSpec((1, B), lambda i: (0, 0)),
            pl.BlockSpec((1, B), lambda i: (0, 0)),
            pl.BlockSpec((8, B), lambda i: (0, 0)),
        ],
        out_shape=[
            jax.ShapeDtypeStruct((1, B), jnp.float32),
            jax.ShapeDtypeStruct((1, B), jnp.float32),
            jax.ShapeDtypeStruct((1, B), jnp.float32),
            jax.ShapeDtypeStruct((8, B), jnp.float32),
        ],
        scratch_shapes=[
            pltpu.VMEM((1, B), jnp.float32),
            pltpu.VMEM((1, B), jnp.float32),
            pltpu.VMEM((1, B), jnp.float32),
        ],
    )(xt)


def _combine_body(C, m_ref, z_ref, s_ref, tail_ref, win_ref, pid_ref, vid_ref,
                  out_ref):
    i = pl.program_id(0)
    n = pl.num_programs(0)
    m = m_ref[...]
    z = z_ref[...]
    s = s_ref[...]
    win = win_ref[...]
    tail = tail_ref[...]
    p3 = pid_ref[...] * 3
    vid = vid_ref[...]

    d0 = lax.broadcasted_iota(jnp.int32, win.shape, 0)
    d2 = lax.broadcasted_iota(jnp.int32, win.shape, 2)
    wcol = jnp.sum(jnp.where(d2 == d0, win, jnp.zeros_like(win)), axis=2)

    start = _slab_start(p3, C)
    rowg = lax.broadcasted_iota(jnp.int32, wcol.shape, 1) + start
    rowt = lax.broadcasted_iota(jnp.int32, tail.shape, 1) + (C - 8)
    A = ((C - _SLAB) >> 3 << 3) + _SLAB
    zs = jnp.zeros_like(wcol)
    zt = jnp.zeros_like(tail)

    def pick(q):
        gw = jnp.sum(jnp.where(rowg == q, wcol, zs), axis=1, keepdims=True)
        gt_ = jnp.sum(jnp.where((rowt == q) & (rowt >= A), tail, zt),
                      axis=1, keepdims=True)
        return gw + gt_

    g0 = pick(p3)
    g1 = pick(p3 + 1)
    g2 = pick(p3 + 2)
    gv = pick(p3 + vid)

    c = m + jnp.log(z)
    ep1 = jnp.exp(g0 - c) + jnp.exp(g1 - c) + jnp.exp(g2 - c)
    ep2 = jnp.exp(gv - c)
    L = s - C * c
    L3 = (g0 + g1 + g2) - 3.0 * c
    lpt = gv - c
    e1 = _ALPHA * (1.0 - ep1)
    e2 = _ALPHA * (1.0 - ep2)
    S = (e1 / (C - 3)) * (L - L3) + 0.5 * e2 * (L3 - lpt) + (1.0 - e1 - e2) * lpt
    contrib = (1.0 - _EPS) * S + (_EPS / C) * L
    bs = jnp.sum(contrib, axis=0, keepdims=True)

    @pl.when(i == 0)
    def _():
        out_ref[...] = jnp.zeros_like(out_ref)

    out_ref[...] += bs

    @pl.when(i == n - 1)
    def _():
        B_total = n * m.shape[0]
        out_ref[...] = out_ref[...] * (-1.0 / B_total)


def _combine(m, z, s, tail_t, win, pids2, vids2, B, C):
    n = B // _RB
    return pl.pallas_call(
        functools.partial(_combine_body, C),
        grid=(n,),
        in_specs=[
            pl.BlockSpec((_RB, 1), lambda i: (i, 0)),
            pl.BlockSpec((_RB, 1), lambda i: (i, 0)),
            pl.BlockSpec((_RB, 1), lambda i: (i, 0)),
            pl.BlockSpec((_RB, 8), lambda i: (i, 0)),
            pl.BlockSpec((_RB, _SLAB, 128), lambda i: (i, 0, 0)),
            pl.BlockSpec((_RB, 1), lambda i: (i, 0)),
            pl.BlockSpec((_RB, 1), lambda i: (i, 0)),
        ],
        out_specs=pl.BlockSpec((1, 1), lambda i: (0, 0)),
        out_shape=jax.ShapeDtypeStruct((1, 1), jnp.float32),
    )(m, z, s, tail_t, win, pids2, vids2)


@jax.jit
def kernel(inputs, pids, vids):
    B, C = inputs.shape
    xt = inputs.T
    pids32 = pids.astype(jnp.int32)
    win = _sc_slabs(xt, pids32, B, C)
    m, z, s, tail = _stream(xt, B, C)
    out = _combine(m.reshape(B, 1), z.reshape(B, 1), s.reshape(B, 1),
                   tail.T, win, pids32.reshape(B, 1),
                   vids.reshape(B, 1).astype(jnp.int32), B, C)
    return out[0, 0]


# speedup vs baseline: 28.4156x; 1.0432x over previous
import functools

import jax
import jax.numpy as jnp
from jax import lax
from jax.experimental import pallas as pl
from jax.experimental.pallas import tpu as pltpu
from jax.experimental.pallas import tpu_sc as plsc

_EPS = 0.1
_ALPHA = 0.2
_CB = 3072
_RB = 128
_SLAB = 16

_NUM_CORES = 2
_NUM_SUBCORES = 16
_NUM_WORKERS = _NUM_CORES * _NUM_SUBCORES


def _slab_start(p3, C):
    return jnp.minimum(p3 >> 3, (C - _SLAB) >> 3) * 8


def _sc_slabs_body(C, BPW, x_hbm, pid_hbm, win_hbm, pid_v, win_v, sem):
    wid = lax.axis_index("s") * _NUM_CORES + lax.axis_index("c")
    base = wid * BPW
    pltpu.sync_copy(pid_hbm.at[pl.ds(base, BPW)], pid_v)
    copies = []
    for r in range(BPW):
        chunk = pid_v[pl.ds((r // 16) * 16, 16)]
        p3 = chunk[r % 16] * 3
        start = _slab_start(p3, C)
        cg = ((base + r) >> 7) * 128
        copies.append(
            pltpu.async_copy(x_hbm.at[pl.ds(start, _SLAB), pl.ds(cg, 128)],
                             win_v.at[r], sem))
    for cp in copies:
        cp.wait()
    pltpu.sync_copy(win_v, win_hbm.at[pl.ds(base, BPW)])


def _sc_slabs(xt, pids, B, C):
    BPW = B // _NUM_WORKERS
    mesh = plsc.VectorSubcoreMesh(core_axis_name="c", subcore_axis_name="s")
    f = functools.partial(
        pl.kernel,
        mesh=mesh,
        out_type=jax.ShapeDtypeStruct((B, _SLAB, 128), jnp.float32),
        scratch_types=[
            pltpu.VMEM((BPW,), jnp.int32),
            pltpu.VMEM((BPW, _SLAB, 128), jnp.float32),
            pltpu.SemaphoreType.DMA,
        ],
    )(functools.partial(_sc_slabs_body, C, BPW))
    return f(xt, pids)


def _stream_body(C, x_ref, m_out, z_out, s_out, tail_out, macc, zacc, sacc):
    i = pl.program_id(0)
    n = pl.num_programs(0)
    x = x_ref[...]
    Cb, B = x.shape

    @pl.when(i == 0)
    def _():
        macc[...] = jnp.full_like(macc, -jnp.inf)
        zacc[...] = jnp.zeros_like(zacc)
        sacc[...] = jnp.zeros_like(sacc)

    def update(xv, xm):
        bm = jnp.max(xm, axis=0, keepdims=True)
        m_new = jnp.maximum(macc[...], bm)
        scale = jnp.exp(macc[...] - m_new)
        zacc[...] = zacc[...] * scale + jnp.sum(
            jnp.exp(xm - m_new), axis=0, keepdims=True)
        sacc[...] += jnp.sum(xv, axis=0, keepdims=True)
        macc[...] = m_new

    @pl.when(i < n - 1)
    def _():
        update(x, x)

    @pl.when(i == n - 1)
    def _():
        row = lax.broadcasted_iota(jnp.int32, x.shape, 0) + i * Cb
        valid = row < C
        xv = jnp.where(valid, x, jnp.zeros_like(x))
        xm = jnp.where(valid, x, jnp.full_like(x, -jnp.inf))
        update(xv, xm)
        m_out[...] = macc[...]
        z_out[...] = zacc[...]
        s_out[...] = sacc[...]
        lo = C - 8 - (n - 1) * Cb
        tail_out[...] = lax.slice(x, (lo, 0), (lo + 8, B))


def _stream(xt, B, C):
    n = (C + _CB - 1) // _CB
    return pl.pallas_call(
        functools.partial(_stream_body, C),
        grid=(n,),
        in_specs=[pl.BlockSpec((_CB, B), lambda i: (i, 0))],
        out_specs=[
            pl.BlockSpec((1, B), lambda i: (0, 0)),
            pl.BlockSpec((1, B), lambda i: (0, 0)),
            pl.BlockSpec((1, B), lambda i: (0, 0)),
            pl.BlockSpec((8, B), lambda i: (0, 0)),
        ],
        out_shape=[
            jax.ShapeDtypeStruct((1, B), jnp.float32),
            jax.ShapeDtypeStruct((1, B), jnp.float32),
            jax.ShapeDtypeStruct((1, B), jnp.float32),
            jax.ShapeDtypeStruct((8, B), jnp.float32),
        ],
        scratch_shapes=[
            pltpu.VMEM((1, B), jnp.float32),
            pltpu.VMEM((1, B), jnp.float32),
            pltpu.VMEM((1, B), jnp.float32),
        ],
    )(xt)


def _combine_body(C, m_ref, z_ref, s_ref, tail_ref, win_ref, pid_ref, vid_ref,
                  out_ref):
    i = pl.program_id(0)
    n = pl.num_programs(0)
    m = m_ref[...]
    z = z_ref[...]
    s = s_ref[...]
    win = win_ref[...]
    tail = tail_ref[...]
    p3 = pid_ref[...] * 3
    vid = vid_ref[...]

    d0 = lax.broadcasted_iota(jnp.int32, win.shape, 0)
    d2 = lax.broadcasted_iota(jnp.int32, win.shape, 2)
    wcol = jnp.sum(jnp.where(d2 == d0, win, jnp.zeros_like(win)), axis=2)

    start = _slab_start(p3, C)
    rowg = lax.broadcasted_iota(jnp.int32, wcol.shape, 1) + start
    rowt = lax.broadcasted_iota(jnp.int32, tail.shape, 1) + (C - 8)
    A = ((C - _SLAB) >> 3 << 3) + _SLAB
    zs = jnp.zeros_like(wcol)
    zt = jnp.zeros_like(tail)

    def pick(q):
        gw = jnp.sum(jnp.where(rowg == q, wcol, zs), axis=1, keepdims=True)
        gt_ = jnp.sum(jnp.where((rowt == q) & (rowt >= A), tail, zt),
                      axis=1, keepdims=True)
        return gw + gt_

    g0 = pick(p3)
    g1 = pick(p3 + 1)
    g2 = pick(p3 + 2)
    gv = pick(p3 + vid)

    c = m + jnp.log(z)
    ep1 = jnp.exp(g0 - c) + jnp.exp(g1 - c) + jnp.exp(g2 - c)
    ep2 = jnp.exp(gv - c)
    L = s - C * c
    L3 = (g0 + g1 + g2) - 3.0 * c
    lpt = gv - c
    e1 = _ALPHA * (1.0 - ep1)
    e2 = _ALPHA * (1.0 - ep2)
    S = (e1 / (C - 3)) * (L - L3) + 0.5 * e2 * (L3 - lpt) + (1.0 - e1 - e2) * lpt
    contrib = (1.0 - _EPS) * S + (_EPS / C) * L
    bs = jnp.sum(contrib, axis=0, keepdims=True)

    @pl.when(i == 0)
    def _():
        out_ref[...] = jnp.zeros_like(out_ref)

    out_ref[...] += bs

    @pl.when(i == n - 1)
    def _():
        B_total = n * m.shape[0]
        out_ref[...] = out_ref[...] * (-1.0 / B_total)


def _combine(m, z, s, tail_t, win, pids2, vids2, B, C):
    n = B // _RB
    return pl.pallas_call(
        functools.partial(_combine_body, C),
        grid=(n,),
        in_specs=[
            pl.BlockSpec((_RB, 1), lambda i: (i, 0)),
            pl.BlockSpec((_RB, 1), lambda i: (i, 0)),
            pl.BlockSpec((_RB, 1), lambda i: (i, 0)),
            pl.BlockSpec((_RB, 8), lambda i: (i, 0)),
            pl.BlockSpec((_RB, _SLAB, 128), lambda i: (i, 0, 0)),
            pl.BlockSpec((_RB, 1), lambda i: (i, 0)),
            pl.BlockSpec((_RB, 1), lambda i: (i, 0)),
        ],
        out_specs=pl.BlockSpec((1, 1), lambda i: (0, 0)),
        out_shape=jax.ShapeDtypeStruct((1, 1), jnp.float32),
    )(m, z, s, tail_t, win, pids2, vids2)


@jax.jit
def kernel(inputs, pids, vids):
    B, C = inputs.shape
    xt = inputs.T
    pids32 = pids.astype(jnp.int32)
    win = _sc_slabs(xt, pids32, B, C)
    m, z, s, tail = _stream(xt, B, C)
    out = _combine(m.reshape(B, 1), z.reshape(B, 1), s.reshape(B, 1),
                   tail.T, win, pids32.reshape(B, 1),
                   vids.reshape(B, 1).astype(jnp.int32), B, C)
    return out[0, 0]
